# baseline (speedup 1.0000x reference)
import functools

import jax
import jax.numpy as jnp
import numpy as np
from jax import lax
from jax.experimental import pallas as pl
from jax.experimental.pallas import tpu as pltpu
from jax.experimental.pallas import tpu_sc as plsc

NUM_INTERVALS = 128
MAX_TIME = 1.0
D_FEAT = 128
N_ROWS = 131072

NC = 2
NS = 16
L = 16
NV = NUM_INTERVALS // L

CHUNK = 128
NCHUNKS = N_ROWS // (NC * NS) // CHUNK


def _sc_body(t_hbm, z_hbm, ind_hbm, dt_hbm, dtind_hbm, tauind_hbm,
             taunext_hbm, z0_hbm, vin, tin, dtbuf, taubuf, indbuf, dtindbuf,
             tauindbuf, taunextbuf, insem, outsem, zsem):
    c = lax.axis_index("c")
    s = lax.axis_index("s")
    wid = c * NS + s
    rows_per = N_ROWS // (NC * NS)
    base = wid * rows_per
    lane = lax.iota(jnp.int32, L)

    def z0_copy(ci):
        row0 = base + ci * CHUNK
        return pltpu.make_async_copy(
            z_hbm.at[pl.ds(row0, CHUNK), pl.ds(0, D_FEAT)],
            z0_hbm.at[pl.ds(row0, CHUNK)], zsem)

    def in_copies(ci, p):
        row0 = base + ci * CHUNK
        return (
            pltpu.make_async_copy(
                z_hbm.at[pl.ds(row0, CHUNK), pl.ds(D_FEAT, NUM_INTERVALS)],
                vin.at[p], insem.at[p]),
            pltpu.make_async_copy(t_hbm.at[pl.ds(row0, CHUNK)], tin.at[p],
                                  insem.at[p]),
        )

    def out_copies(ci, p):
        row0 = base + ci * CHUNK
        dst = pl.ds(row0, CHUNK)
        return (
            pltpu.make_async_copy(dtbuf.at[p], dt_hbm.at[dst], outsem.at[p]),
            pltpu.make_async_copy(indbuf.at[p], ind_hbm.at[dst], outsem.at[p]),
            pltpu.make_async_copy(dtindbuf.at[p], dtind_hbm.at[dst],
                                  outsem.at[p]),
            pltpu.make_async_copy(tauindbuf.at[p], tauind_hbm.at[dst],
                                  outsem.at[p]),
            pltpu.make_async_copy(taunextbuf.at[p], taunext_hbm.at[dst],
                                  outsem.at[p]),
        )

    for cp in in_copies(0, 0):
        cp.start()
    for cp in in_copies(1, 1):
        cp.start()

    def chunk_body(ci, carry):
        p = jnp.bitwise_and(ci, 1)
        z0_copy(ci).start()
        for cp in in_copies(ci, p):
            cp.wait()

        @pl.when(ci >= 2)
        def _():
            for cp in out_copies(ci - 2, p):
                cp.wait()

        def group_body(g, gcarry):
            ind_acc = jnp.zeros((L,), jnp.int32)
            t16 = tin[p, pl.ds(g * L, L)]
            for j in range(L):
                r = g * L + j
                tj = t16[j]
                e = []
                cume = []
                for i in range(NV):
                    ei = jnp.exp(vin[p, r, pl.ds(i * L, L)])
                    e.append(ei)
                    cume.append(plsc.cumsum(ei))
                prefix = []
                tot = np.float32(0.0)
                for i in range(NV):
                    prefix.append(tot)
                    tot = tot + cume[i][L - 1]
                inv = 1.0 / jnp.broadcast_to(tot, (L,))
                cnt = jnp.zeros((L,), jnp.int32)
                for i in range(NV):
                    dtbuf[p, r, pl.ds(i * L, L)] = e[i] * inv
                    taui = (cume[i] + prefix[i]) * inv
                    taubuf[j, pl.ds(i * L, L)] = taui
                    m = taui < tj
                    if i == NV - 1:
                        m = m & (lane < L - 1)
                    cnt = cnt + m.astype(jnp.int32)
                indj = jnp.sum(cnt)
                ind_acc = ind_acc + jnp.where(lane == j, indj, 0)
            rows16 = g * L + lane
            p16 = jnp.broadcast_to(p, (L,))
            dtind16 = plsc.load_gather(dtbuf, [p16, rows16, ind_acc])
            taunext16 = plsc.load_gather(taubuf, [lane, ind_acc])
            indbuf[p, pl.ds(g * L, L)] = ind_acc
            dtindbuf[p, pl.ds(g * L, L)] = dtind16
            taunextbuf[p, pl.ds(g * L, L)] = taunext16
            tauindbuf[p, pl.ds(g * L, L)] = taunext16 - dtind16
            return gcarry

        lax.fori_loop(0, CHUNK // L, group_body, 0)

        for cp in out_copies(ci, p):
            cp.start()

        @pl.when(ci + 2 < NCHUNKS)
        def _():
            for cp in in_copies(ci + 2, p):
                cp.start()

        return carry

    lax.fori_loop(0, NCHUNKS, chunk_body, 0)

    def z0_drain(ci, carry):
        z0_copy(ci).wait()
        return carry

    lax.fori_loop(0, NCHUNKS, z0_drain, 0)

    for cp in out_copies(NCHUNKS - 2, 0):
        cp.wait()
    for cp in out_copies(NCHUNKS - 1, 1):
        cp.wait()


@jax.jit
def kernel(t, z):
    n = t.shape[0]
    mesh = plsc.VectorSubcoreMesh(core_axis_name="c", subcore_axis_name="s")
    out_type = (
        jax.ShapeDtypeStruct((n,), jnp.int32),
        jax.ShapeDtypeStruct((n, NUM_INTERVALS), jnp.float32),
        jax.ShapeDtypeStruct((n,), jnp.float32),
        jax.ShapeDtypeStruct((n,), jnp.float32),
        jax.ShapeDtypeStruct((n,), jnp.float32),
        jax.ShapeDtypeStruct((n, D_FEAT), jnp.float32),
    )
    scratch = [
        pltpu.VMEM((2, CHUNK, NUM_INTERVALS), jnp.float32),
        pltpu.VMEM((2, CHUNK), jnp.float32),
        pltpu.VMEM((2, CHUNK, NUM_INTERVALS), jnp.float32),
        pltpu.VMEM((L, NUM_INTERVALS), jnp.float32),
        pltpu.VMEM((2, CHUNK), jnp.int32),
        pltpu.VMEM((2, CHUNK), jnp.float32),
        pltpu.VMEM((2, CHUNK), jnp.float32),
        pltpu.VMEM((2, CHUNK), jnp.float32),
        pltpu.SemaphoreType.DMA((2,)),
        pltpu.SemaphoreType.DMA((2,)),
        pltpu.SemaphoreType.DMA,
    ]
    ind, dt, dt_ind, tau_ind, tau_next, z0 = pl.kernel(
        _sc_body,
        out_type=out_type,
        mesh=mesh,
        scratch_types=scratch,
        compiler_params=pltpu.CompilerParams(needs_layout_passes=False),
    )(t, z)
    return (ind, dt, dt_ind, tau_ind, tau_next, z0)


# speedup vs baseline: 8.7734x; 8.7734x over previous
import functools

import jax
import jax.numpy as jnp
import numpy as np
from jax import lax
from jax.experimental import pallas as pl
from jax.experimental.pallas import tpu as pltpu
from jax.experimental.pallas import tpu_sc as plsc

NUM_INTERVALS = 128
MAX_TIME = 1.0
D_FEAT = 128
N_ROWS = 131072

NC = 2
NS = 16
L = 16
NV = NUM_INTERVALS // L

CHUNK = 128
NCHUNKS = N_ROWS // (NC * NS) // CHUNK


def _sc_body(t_hbm, z_hbm, ind_hbm, dt_hbm, dtind_hbm, tauind_hbm,
             taunext_hbm, vin, tin, dtbuf, taubuf, indbuf, dtindbuf,
             tauindbuf, taunextbuf, insem, outsem):
    c = lax.axis_index("c")
    s = lax.axis_index("s")
    wid = c * NS + s
    rows_per = N_ROWS // (NC * NS)
    base = wid * rows_per
    lane = lax.iota(jnp.int32, L)

    def in_copies(ci, p):
        row0 = base + ci * CHUNK
        return (
            pltpu.make_async_copy(
                z_hbm.at[pl.ds(row0, CHUNK), pl.ds(D_FEAT, NUM_INTERVALS)],
                vin.at[p], insem.at[p]),
            pltpu.make_async_copy(t_hbm.at[pl.ds(row0, CHUNK)], tin.at[p],
                                  insem.at[p]),
        )

    def out_copies(ci, p):
        row0 = base + ci * CHUNK
        dst = pl.ds(row0, CHUNK)
        return (
            pltpu.make_async_copy(dtbuf.at[p], dt_hbm.at[dst], outsem.at[p]),
            pltpu.make_async_copy(indbuf.at[p], ind_hbm.at[dst], outsem.at[p]),
            pltpu.make_async_copy(dtindbuf.at[p], dtind_hbm.at[dst],
                                  outsem.at[p]),
            pltpu.make_async_copy(tauindbuf.at[p], tauind_hbm.at[dst],
                                  outsem.at[p]),
            pltpu.make_async_copy(taunextbuf.at[p], taunext_hbm.at[dst],
                                  outsem.at[p]),
        )

    for cp in in_copies(0, 0):
        cp.start()
    for cp in in_copies(1, 1):
        cp.start()

    def chunk_body(ci, carry):
        p = jnp.bitwise_and(ci, 1)
        for cp in in_copies(ci, p):
            cp.wait()

        @pl.when(ci >= 2)
        def _():
            for cp in out_copies(ci - 2, p):
                cp.wait()

        gdn = lax.GatherDimensionNumbers(
            offset_dims=(), collapsed_slice_dims=(0,), start_index_map=(0,))

        def splat_last(x):
            idx = jnp.full((L, 1), L - 1, jnp.int32)
            return lax.gather(x, idx, gdn, (1,),
                              mode=lax.GatherScatterMode.PROMISE_IN_BOUNDS)

        def group_body(g, gcarry):
            ind_acc = jnp.zeros((L,), jnp.int32)
            t16 = tin[p, pl.ds(g * L, L)]
            for j in range(L):
                r = g * L + j
                tj = t16[j]
                e = []
                cume = []
                for i in range(NV):
                    ei = jnp.exp(vin[p, r, pl.ds(i * L, L)])
                    e.append(ei)
                    cume.append(plsc.cumsum(ei))
                prefv = []
                pref = jnp.zeros((L,), jnp.float32)
                for i in range(NV):
                    prefv.append(pref)
                    pref = pref + splat_last(cume[i])
                inv = 1.0 / pref
                cnt = jnp.zeros((L,), jnp.int32)
                for i in range(NV):
                    dtbuf[p, r, pl.ds(i * L, L)] = e[i] * inv
                    taui = (cume[i] + prefv[i]) * inv
                    taubuf[j, pl.ds(i * L, L)] = taui
                    m = taui < tj
                    if i == NV - 1:
                        m = m & (lane < L - 1)
                    cnt = cnt + m.astype(jnp.int32)
                indj = jnp.sum(cnt)
                ind_acc = ind_acc + jnp.where(lane == j, indj, 0)
            rows16 = g * L + lane
            p16 = jnp.broadcast_to(p, (L,))
            dtind16 = plsc.load_gather(dtbuf, [p16, rows16, ind_acc])
            taunext16 = plsc.load_gather(taubuf, [lane, ind_acc])
            indbuf[p, pl.ds(g * L, L)] = ind_acc
            dtindbuf[p, pl.ds(g * L, L)] = dtind16
            taunextbuf[p, pl.ds(g * L, L)] = taunext16
            tauindbuf[p, pl.ds(g * L, L)] = taunext16 - dtind16
            return gcarry

        lax.fori_loop(0, CHUNK // L, group_body, 0)

        for cp in out_copies(ci, p):
            cp.start()

        @pl.when(ci + 2 < NCHUNKS)
        def _():
            for cp in in_copies(ci + 2, p):
                cp.start()

        return carry

    lax.fori_loop(0, NCHUNKS, chunk_body, 0)

    for cp in out_copies(NCHUNKS - 2, 0):
        cp.wait()
    for cp in out_copies(NCHUNKS - 1, 1):
        cp.wait()


@jax.jit
def kernel(t, z):
    n = t.shape[0]
    mesh = plsc.VectorSubcoreMesh(core_axis_name="c", subcore_axis_name="s")
    out_type = (
        jax.ShapeDtypeStruct((n,), jnp.int32),
        jax.ShapeDtypeStruct((n, NUM_INTERVALS), jnp.float32),
        jax.ShapeDtypeStruct((n,), jnp.float32),
        jax.ShapeDtypeStruct((n,), jnp.float32),
        jax.ShapeDtypeStruct((n,), jnp.float32),
    )
    scratch = [
        pltpu.VMEM((2, CHUNK, NUM_INTERVALS), jnp.float32),
        pltpu.VMEM((2, CHUNK), jnp.float32),
        pltpu.VMEM((2, CHUNK, NUM_INTERVALS), jnp.float32),
        pltpu.VMEM((L, NUM_INTERVALS), jnp.float32),
        pltpu.VMEM((2, CHUNK), jnp.int32),
        pltpu.VMEM((2, CHUNK), jnp.float32),
        pltpu.VMEM((2, CHUNK), jnp.float32),
        pltpu.VMEM((2, CHUNK), jnp.float32),
        pltpu.SemaphoreType.DMA((2,)),
        pltpu.SemaphoreType.DMA((2,)),
    ]
    ind, dt, dt_ind, tau_ind, tau_next = pl.kernel(
        _sc_body,
        out_type=out_type,
        mesh=mesh,
        scratch_types=scratch,
        compiler_params=pltpu.CompilerParams(needs_layout_passes=False),
    )(t, z)
    z0 = z[:, :D_FEAT]
    return (ind, dt, dt_ind, tau_ind, tau_next, z0)


# speedup vs baseline: 8.8686x; 1.0108x over previous
import functools

import jax
import jax.numpy as jnp
import numpy as np
from jax import lax
from jax.experimental import pallas as pl
from jax.experimental.pallas import tpu as pltpu
from jax.experimental.pallas import tpu_sc as plsc

NUM_INTERVALS = 128
MAX_TIME = 1.0
D_FEAT = 128
N_ROWS = 131072

NC = 2
NS = 16
L = 16
NV = NUM_INTERVALS // L

CHUNK = 128
NCHUNKS = N_ROWS // (NC * NS) // CHUNK


def _sc_body(t_hbm, z_hbm, ind_hbm, dt_hbm, dtind_hbm, tauind_hbm,
             taunext_hbm, vin, tin, dtbuf, taubuf, indbuf, dtindbuf,
             tauindbuf, taunextbuf, insem, outsem):
    c = lax.axis_index("c")
    s = lax.axis_index("s")
    wid = c * NS + s
    rows_per = N_ROWS // (NC * NS)
    base = wid * rows_per
    lane = lax.iota(jnp.int32, L)

    def in_copies(ci, p):
        row0 = base + ci * CHUNK
        return (
            pltpu.make_async_copy(
                z_hbm.at[pl.ds(row0, CHUNK), pl.ds(D_FEAT, NUM_INTERVALS)],
                vin.at[p], insem.at[p]),
            pltpu.make_async_copy(t_hbm.at[pl.ds(row0, CHUNK)], tin.at[p],
                                  insem.at[p]),
        )

    def out_copies(ci, p):
        row0 = base + ci * CHUNK
        dst = pl.ds(row0, CHUNK)
        return (
            pltpu.make_async_copy(dtbuf.at[p], dt_hbm.at[dst], outsem.at[p]),
            pltpu.make_async_copy(indbuf.at[p], ind_hbm.at[dst], outsem.at[p]),
            pltpu.make_async_copy(dtindbuf.at[p], dtind_hbm.at[dst],
                                  outsem.at[p]),
            pltpu.make_async_copy(tauindbuf.at[p], tauind_hbm.at[dst],
                                  outsem.at[p]),
            pltpu.make_async_copy(taunextbuf.at[p], taunext_hbm.at[dst],
                                  outsem.at[p]),
        )

    for cp in in_copies(0, 0):
        cp.start()
    for cp in in_copies(1, 1):
        cp.start()

    def chunk_body(ci, carry):
        p = jnp.bitwise_and(ci, 1)
        for cp in in_copies(ci, p):
            cp.wait()

        @pl.when(ci >= 2)
        def _():
            for cp in out_copies(ci - 2, p):
                cp.wait()

        gdn = lax.GatherDimensionNumbers(
            offset_dims=(), collapsed_slice_dims=(0,), start_index_map=(0,))

        def splat_at(x, k):
            idx = jnp.full((L, 1), k, jnp.int32)
            return lax.gather(x, idx, gdn, (1,),
                              mode=lax.GatherScatterMode.PROMISE_IN_BOUNDS)

        def splat_last(x):
            return splat_at(x, L - 1)

        def group_body(g, gcarry):
            ind_acc = jnp.zeros((L,), jnp.int32)
            t16 = tin[p, pl.ds(g * L, L)]
            for j in range(L):
                r = g * L + j
                tj = splat_at(t16, j)
                e = []
                cume = []
                for i in range(NV):
                    ei = jnp.exp(vin[p, r, pl.ds(i * L, L)])
                    e.append(ei)
                    cume.append(plsc.cumsum(ei))
                prefv = []
                pref = jnp.zeros((L,), jnp.float32)
                for i in range(NV):
                    prefv.append(pref)
                    pref = pref + splat_last(cume[i])
                inv = 1.0 / pref
                cnt = jnp.zeros((L,), jnp.int32)
                for i in range(NV):
                    dtbuf[p, r, pl.ds(i * L, L)] = e[i] * inv
                    taui = (cume[i] + prefv[i]) * inv
                    taubuf[j, pl.ds(i * L, L)] = taui
                    m = taui < tj
                    if i == NV - 1:
                        m = m & (lane < L - 1)
                    cnt = cnt + m.astype(jnp.int32)
                indj = splat_last(plsc.cumsum(cnt))
                ind_acc = ind_acc + jnp.where(lane == j, indj, 0)
            rows16 = g * L + lane
            p16 = jnp.broadcast_to(p, (L,))
            dtind16 = plsc.load_gather(dtbuf, [p16, rows16, ind_acc])
            taunext16 = plsc.load_gather(taubuf, [lane, ind_acc])
            indbuf[p, pl.ds(g * L, L)] = ind_acc
            dtindbuf[p, pl.ds(g * L, L)] = dtind16
            taunextbuf[p, pl.ds(g * L, L)] = taunext16
            tauindbuf[p, pl.ds(g * L, L)] = taunext16 - dtind16
            return gcarry

        lax.fori_loop(0, CHUNK // L, group_body, 0)

        for cp in out_copies(ci, p):
            cp.start()

        @pl.when(ci + 2 < NCHUNKS)
        def _():
            for cp in in_copies(ci + 2, p):
                cp.start()

        return carry

    lax.fori_loop(0, NCHUNKS, chunk_body, 0)

    for cp in out_copies(NCHUNKS - 2, 0):
        cp.wait()
    for cp in out_copies(NCHUNKS - 1, 1):
        cp.wait()


@jax.jit
def kernel(t, z):
    n = t.shape[0]
    mesh = plsc.VectorSubcoreMesh(core_axis_name="c", subcore_axis_name="s")
    out_type = (
        jax.ShapeDtypeStruct((n,), jnp.int32),
        jax.ShapeDtypeStruct((n, NUM_INTERVALS), jnp.float32),
        jax.ShapeDtypeStruct((n,), jnp.float32),
        jax.ShapeDtypeStruct((n,), jnp.float32),
        jax.ShapeDtypeStruct((n,), jnp.float32),
    )
    scratch = [
        pltpu.VMEM((2, CHUNK, NUM_INTERVALS), jnp.float32),
        pltpu.VMEM((2, CHUNK), jnp.float32),
        pltpu.VMEM((2, CHUNK, NUM_INTERVALS), jnp.float32),
        pltpu.VMEM((L, NUM_INTERVALS), jnp.float32),
        pltpu.VMEM((2, CHUNK), jnp.int32),
        pltpu.VMEM((2, CHUNK), jnp.float32),
        pltpu.VMEM((2, CHUNK), jnp.float32),
        pltpu.VMEM((2, CHUNK), jnp.float32),
        pltpu.SemaphoreType.DMA((2,)),
        pltpu.SemaphoreType.DMA((2,)),
    ]
    ind, dt, dt_ind, tau_ind, tau_next = pl.kernel(
        _sc_body,
        out_type=out_type,
        mesh=mesh,
        scratch_types=scratch,
        compiler_params=pltpu.CompilerParams(needs_layout_passes=False),
    )(t, z)
    z0 = z[:, :D_FEAT]
    return (ind, dt, dt_ind, tau_ind, tau_next, z0)


# speedup vs baseline: 12.4004x; 1.3982x over previous
import functools

import jax
import jax.numpy as jnp
import numpy as np
from jax import lax
from jax.experimental import pallas as pl
from jax.experimental.pallas import tpu as pltpu
from jax.experimental.pallas import tpu_sc as plsc

NUM_INTERVALS = 128
MAX_TIME = 1.0
D_FEAT = 128
N_ROWS = 131072

NC = 2
NS = 16
L = 16
NV = NUM_INTERVALS // L

CHUNK = 128
NCHUNKS = N_ROWS // (NC * NS) // CHUNK


def _sc_body(t_hbm, z_hbm, ind_hbm, dt_hbm, dtind_hbm, tauind_hbm,
             taunext_hbm, vin, tin, dtbuf, taubuf, indbuf, dtindbuf,
             tauindbuf, taunextbuf, insem, outsem):
    c = lax.axis_index("c")
    s = lax.axis_index("s")
    wid = c * NS + s
    rows_per = N_ROWS // (NC * NS)
    base = wid * rows_per
    lane = lax.iota(jnp.int32, L)

    def in_copies(ci, p):
        row0 = base + ci * CHUNK
        return (
            pltpu.make_async_copy(
                z_hbm.at[pl.ds(row0, CHUNK), pl.ds(D_FEAT, NUM_INTERVALS)],
                vin.at[p], insem.at[p]),
            pltpu.make_async_copy(t_hbm.at[pl.ds(row0, CHUNK)], tin.at[p],
                                  insem.at[p]),
        )

    def out_copies(ci, p):
        row0 = base + ci * CHUNK
        dst = pl.ds(row0, CHUNK)
        return (
            pltpu.make_async_copy(dtbuf.at[p], dt_hbm.at[dst], outsem.at[p]),
            pltpu.make_async_copy(indbuf.at[p], ind_hbm.at[dst], outsem.at[p]),
            pltpu.make_async_copy(dtindbuf.at[p], dtind_hbm.at[dst],
                                  outsem.at[p]),
            pltpu.make_async_copy(tauindbuf.at[p], tauind_hbm.at[dst],
                                  outsem.at[p]),
            pltpu.make_async_copy(taunextbuf.at[p], taunext_hbm.at[dst],
                                  outsem.at[p]),
        )

    for cp in in_copies(0, 0):
        cp.start()
    for cp in in_copies(1, 1):
        cp.start()

    def chunk_body(ci, carry):
        p = jnp.bitwise_and(ci, 1)
        for cp in in_copies(ci, p):
            cp.wait()

        @pl.when(ci >= 2)
        def _():
            for cp in out_copies(ci - 2, p):
                cp.wait()

        gdn = lax.GatherDimensionNumbers(
            offset_dims=(), collapsed_slice_dims=(0,), start_index_map=(0,))

        def splat_at(x, k):
            idx = jnp.full((L, 1), k, jnp.int32)
            return lax.gather(x, idx, gdn, (1,),
                              mode=lax.GatherScatterMode.PROMISE_IN_BOUNDS)

        def splat_last(x):
            return splat_at(x, L - 1)

        def group_body(g, gcarry):
            ind_acc = jnp.zeros((L,), jnp.int32)
            t16 = tin[p, pl.ds(g * L, L)]
            for jj in range(0, L, 2):
                rows = (g * L + jj, g * L + jj + 1)
                tjs = (splat_at(t16, jj), splat_at(t16, jj + 1))
                e = ([], [])
                cume = ([], [])
                for i in range(NV):
                    for w in (0, 1):
                        ei = jnp.exp(vin[p, rows[w], pl.ds(i * L, L)])
                        e[w].append(ei)
                        cume[w].append(plsc.cumsum(ei))
                prefv = ([], [])
                pref = [jnp.zeros((L,), jnp.float32)] * 2
                for i in range(NV):
                    for w in (0, 1):
                        prefv[w].append(pref[w])
                        pref[w] = pref[w] + splat_last(cume[w][i])
                invs = (1.0 / pref[0], 1.0 / pref[1])
                cnt = [jnp.zeros((L,), jnp.int32)] * 2
                for i in range(NV):
                    for w in (0, 1):
                        dtbuf[p, rows[w], pl.ds(i * L, L)] = e[w][i] * invs[w]
                        taui = (cume[w][i] + prefv[w][i]) * invs[w]
                        taubuf[jj + w, pl.ds(i * L, L)] = taui
                        m = taui < tjs[w]
                        if i == NV - 1:
                            m = m & (lane < L - 1)
                        cnt[w] = cnt[w] + m.astype(jnp.int32)
                for w in (0, 1):
                    indj = splat_last(plsc.cumsum(cnt[w]))
                    ind_acc = ind_acc + jnp.where(lane == jj + w, indj, 0)
            rows16 = g * L + lane
            p16 = jnp.broadcast_to(p, (L,))
            dtind16 = plsc.load_gather(dtbuf, [p16, rows16, ind_acc])
            taunext16 = plsc.load_gather(taubuf, [lane, ind_acc])
            indbuf[p, pl.ds(g * L, L)] = ind_acc
            dtindbuf[p, pl.ds(g * L, L)] = dtind16
            taunextbuf[p, pl.ds(g * L, L)] = taunext16
            tauindbuf[p, pl.ds(g * L, L)] = taunext16 - dtind16
            return gcarry

        lax.fori_loop(0, CHUNK // L, group_body, 0)

        for cp in out_copies(ci, p):
            cp.start()

        @pl.when(ci + 2 < NCHUNKS)
        def _():
            for cp in in_copies(ci + 2, p):
                cp.start()

        return carry

    lax.fori_loop(0, NCHUNKS, chunk_body, 0)

    for cp in out_copies(NCHUNKS - 2, 0):
        cp.wait()
    for cp in out_copies(NCHUNKS - 1, 1):
        cp.wait()


@jax.jit
def kernel(t, z):
    n = t.shape[0]
    mesh = plsc.VectorSubcoreMesh(core_axis_name="c", subcore_axis_name="s")
    out_type = (
        jax.ShapeDtypeStruct((n,), jnp.int32),
        jax.ShapeDtypeStruct((n, NUM_INTERVALS), jnp.float32),
        jax.ShapeDtypeStruct((n,), jnp.float32),
        jax.ShapeDtypeStruct((n,), jnp.float32),
        jax.ShapeDtypeStruct((n,), jnp.float32),
    )
    scratch = [
        pltpu.VMEM((2, CHUNK, NUM_INTERVALS), jnp.float32),
        pltpu.VMEM((2, CHUNK), jnp.float32),
        pltpu.VMEM((2, CHUNK, NUM_INTERVALS), jnp.float32),
        pltpu.VMEM((L, NUM_INTERVALS), jnp.float32),
        pltpu.VMEM((2, CHUNK), jnp.int32),
        pltpu.VMEM((2, CHUNK), jnp.float32),
        pltpu.VMEM((2, CHUNK), jnp.float32),
        pltpu.VMEM((2, CHUNK), jnp.float32),
        pltpu.SemaphoreType.DMA((2,)),
        pltpu.SemaphoreType.DMA((2,)),
    ]
    ind, dt, dt_ind, tau_ind, tau_next = pl.kernel(
        _sc_body,
        out_type=out_type,
        mesh=mesh,
        scratch_types=scratch,
        compiler_params=pltpu.CompilerParams(needs_layout_passes=False),
    )(t, z)
    z0 = z[:, :D_FEAT]
    return (ind, dt, dt_ind, tau_ind, tau_next, z0)


# speedup vs baseline: 12.6618x; 1.0211x over previous
import functools

import jax
import jax.numpy as jnp
import numpy as np
from jax import lax
from jax.experimental import pallas as pl
from jax.experimental.pallas import tpu as pltpu
from jax.experimental.pallas import tpu_sc as plsc

NUM_INTERVALS = 128
MAX_TIME = 1.0
D_FEAT = 128
N_ROWS = 131072

NC = 2
NS = 16
L = 16
NV = NUM_INTERVALS // L

CHUNK = 128
NCHUNKS = N_ROWS // (NC * NS) // CHUNK


def _sc_body(t_hbm, z_hbm, ind_hbm, dt_hbm, dtind_hbm, tauind_hbm,
             taunext_hbm, vin, tin, dtbuf, taubuf, indbuf, dtindbuf,
             tauindbuf, taunextbuf, insem, outsem):
    c = lax.axis_index("c")
    s = lax.axis_index("s")
    wid = c * NS + s
    rows_per = N_ROWS // (NC * NS)
    base = wid * rows_per
    lane = lax.iota(jnp.int32, L)

    def in_copies(ci, p):
        row0 = base + ci * CHUNK
        return (
            pltpu.make_async_copy(
                z_hbm.at[pl.ds(row0, CHUNK), pl.ds(D_FEAT, NUM_INTERVALS)],
                vin.at[p], insem.at[p]),
            pltpu.make_async_copy(t_hbm.at[pl.ds(row0, CHUNK)], tin.at[p],
                                  insem.at[p]),
        )

    def out_copies(ci, p):
        row0 = base + ci * CHUNK
        dst = pl.ds(row0, CHUNK)
        return (
            pltpu.make_async_copy(dtbuf.at[p], dt_hbm.at[dst], outsem.at[p]),
            pltpu.make_async_copy(indbuf.at[p], ind_hbm.at[dst], outsem.at[p]),
            pltpu.make_async_copy(dtindbuf.at[p], dtind_hbm.at[dst],
                                  outsem.at[p]),
            pltpu.make_async_copy(tauindbuf.at[p], tauind_hbm.at[dst],
                                  outsem.at[p]),
            pltpu.make_async_copy(taunextbuf.at[p], taunext_hbm.at[dst],
                                  outsem.at[p]),
        )

    for cp in in_copies(0, 0):
        cp.start()
    for cp in in_copies(1, 1):
        cp.start()

    def chunk_body(ci, carry):
        p = jnp.bitwise_and(ci, 1)
        for cp in in_copies(ci, p):
            cp.wait()

        @pl.when(ci >= 2)
        def _():
            for cp in out_copies(ci - 2, p):
                cp.wait()

        gdn = lax.GatherDimensionNumbers(
            offset_dims=(), collapsed_slice_dims=(0,), start_index_map=(0,))

        def splat_at(x, k):
            idx = jnp.full((L, 1), k, jnp.int32)
            return lax.gather(x, idx, gdn, (1,),
                              mode=lax.GatherScatterMode.PROMISE_IN_BOUNDS)

        def splat_last(x):
            return splat_at(x, L - 1)

        def group_body(g, gcarry):
            ind_acc = jnp.zeros((L,), jnp.int32)
            t16 = tin[p, pl.ds(g * L, L)]
            def do_rows(js, ind_acc):
                nw = len(js)
                ws = range(nw)
                rows = [g * L + j for j in js]
                tjs = [splat_at(t16, j) for j in js]
                e = [[] for _ in ws]
                cume = [[] for _ in ws]
                for i in range(NV):
                    for w in ws:
                        ei = jnp.exp(vin[p, rows[w], pl.ds(i * L, L)])
                        e[w].append(ei)
                        cume[w].append(plsc.cumsum(ei))
                prefv = [[] for _ in ws]
                pref = [jnp.zeros((L,), jnp.float32)] * nw
                for i in range(NV):
                    for w in ws:
                        prefv[w].append(pref[w])
                        pref[w] = pref[w] + splat_last(cume[w][i])
                invs = [1.0 / pref[w] for w in ws]
                cnt = [jnp.zeros((L,), jnp.int32)] * nw
                for i in range(NV):
                    for w in ws:
                        dtbuf[p, rows[w], pl.ds(i * L, L)] = e[w][i] * invs[w]
                        taui = (cume[w][i] + prefv[w][i]) * invs[w]
                        taubuf[js[w], pl.ds(i * L, L)] = taui
                        m = taui < tjs[w]
                        if i == NV - 1:
                            m = m & (lane < L - 1)
                        cnt[w] = cnt[w] + m.astype(jnp.int32)
                for w in ws:
                    indj = splat_last(plsc.cumsum(cnt[w]))
                    ind_acc = ind_acc + jnp.where(lane == js[w], indj, 0)
                return ind_acc

            for b in (0, 3, 6, 9, 12):
                ind_acc = do_rows((b, b + 1, b + 2), ind_acc)
            ind_acc = do_rows((15,), ind_acc)
            rows16 = g * L + lane
            p16 = jnp.broadcast_to(p, (L,))
            dtind16 = plsc.load_gather(dtbuf, [p16, rows16, ind_acc])
            taunext16 = plsc.load_gather(taubuf, [lane, ind_acc])
            indbuf[p, pl.ds(g * L, L)] = ind_acc
            dtindbuf[p, pl.ds(g * L, L)] = dtind16
            taunextbuf[p, pl.ds(g * L, L)] = taunext16
            tauindbuf[p, pl.ds(g * L, L)] = taunext16 - dtind16
            return gcarry

        lax.fori_loop(0, CHUNK // L, group_body, 0)

        for cp in out_copies(ci, p):
            cp.start()

        @pl.when(ci + 2 < NCHUNKS)
        def _():
            for cp in in_copies(ci + 2, p):
                cp.start()

        return carry

    lax.fori_loop(0, NCHUNKS, chunk_body, 0)

    for cp in out_copies(NCHUNKS - 2, 0):
        cp.wait()
    for cp in out_copies(NCHUNKS - 1, 1):
        cp.wait()


@jax.jit
def kernel(t, z):
    n = t.shape[0]
    mesh = plsc.VectorSubcoreMesh(core_axis_name="c", subcore_axis_name="s")
    out_type = (
        jax.ShapeDtypeStruct((n,), jnp.int32),
        jax.ShapeDtypeStruct((n, NUM_INTERVALS), jnp.float32),
        jax.ShapeDtypeStruct((n,), jnp.float32),
        jax.ShapeDtypeStruct((n,), jnp.float32),
        jax.ShapeDtypeStruct((n,), jnp.float32),
    )
    scratch = [
        pltpu.VMEM((2, CHUNK, NUM_INTERVALS), jnp.float32),
        pltpu.VMEM((2, CHUNK), jnp.float32),
        pltpu.VMEM((2, CHUNK, NUM_INTERVALS), jnp.float32),
        pltpu.VMEM((L, NUM_INTERVALS), jnp.float32),
        pltpu.VMEM((2, CHUNK), jnp.int32),
        pltpu.VMEM((2, CHUNK), jnp.float32),
        pltpu.VMEM((2, CHUNK), jnp.float32),
        pltpu.VMEM((2, CHUNK), jnp.float32),
        pltpu.SemaphoreType.DMA((2,)),
        pltpu.SemaphoreType.DMA((2,)),
    ]
    ind, dt, dt_ind, tau_ind, tau_next = pl.kernel(
        _sc_body,
        out_type=out_type,
        mesh=mesh,
        scratch_types=scratch,
        compiler_params=pltpu.CompilerParams(needs_layout_passes=False),
    )(t, z)
    z0 = z[:, :D_FEAT]
    return (ind, dt, dt_ind, tau_ind, tau_next, z0)


# speedup vs baseline: 12.6709x; 1.0007x over previous
import jax
import jax.numpy as jnp
from jax import lax
from jax.experimental import pallas as pl
from jax.experimental.pallas import tpu as pltpu
from jax.experimental.pallas import tpu_sc as plsc

NUM_INTERVALS = 128
MAX_TIME = 1.0
D_FEAT = 128
N_ROWS = 131072

NC = 2
NS = 16
L = 16
NV = NUM_INTERVALS // L

CHUNK = 128
NCHUNKS = N_ROWS // (NC * NS) // CHUNK


def _sc_body(t_hbm, z_hbm, ind_hbm, dt_hbm, dtind_hbm, tauind_hbm,
             taunext_hbm, vin, tin, dtbuf, taubuf, indbuf, dtindbuf,
             tauindbuf, taunextbuf, insem, outsem):
    c = lax.axis_index("c")
    s = lax.axis_index("s")
    wid = c * NS + s
    rows_per = N_ROWS // (NC * NS)
    base = wid * rows_per
    lane = lax.iota(jnp.int32, L)

    def in_copies(ci, p):
        row0 = base + ci * CHUNK
        return (
            pltpu.make_async_copy(
                z_hbm.at[pl.ds(row0, CHUNK), pl.ds(D_FEAT, NUM_INTERVALS)],
                vin.at[p], insem.at[p]),
            pltpu.make_async_copy(t_hbm.at[pl.ds(row0, CHUNK)], tin.at[p],
                                  insem.at[p]),
        )

    def out_copies(ci, p):
        row0 = base + ci * CHUNK
        dst = pl.ds(row0, CHUNK)
        return (
            pltpu.make_async_copy(dtbuf.at[p], dt_hbm.at[dst], outsem.at[p]),
            pltpu.make_async_copy(indbuf.at[p], ind_hbm.at[dst], outsem.at[p]),
            pltpu.make_async_copy(dtindbuf.at[p], dtind_hbm.at[dst],
                                  outsem.at[p]),
            pltpu.make_async_copy(tauindbuf.at[p], tauind_hbm.at[dst],
                                  outsem.at[p]),
            pltpu.make_async_copy(taunextbuf.at[p], taunext_hbm.at[dst],
                                  outsem.at[p]),
        )

    for cp in in_copies(0, 0):
        cp.start()
    for cp in in_copies(1, 1):
        cp.start()

    def chunk_body(ci, carry):
        p = jnp.bitwise_and(ci, 1)
        for cp in in_copies(ci, p):
            cp.wait()

        @pl.when(ci >= 2)
        def _():
            for cp in out_copies(ci - 2, p):
                cp.wait()

        gdn = lax.GatherDimensionNumbers(
            offset_dims=(), collapsed_slice_dims=(0,), start_index_map=(0,))

        def splat_at(x, k):
            idx = jnp.full((L, 1), k, jnp.int32)
            return lax.gather(x, idx, gdn, (1,),
                              mode=lax.GatherScatterMode.PROMISE_IN_BOUNDS)

        def splat_last(x):
            return splat_at(x, L - 1)

        def group_body(g, gcarry):
            ind_acc = jnp.zeros((L,), jnp.int32)
            t16 = tin[p, pl.ds(g * L, L)]
            def do_rows(js, ind_acc):
                nw = len(js)
                ws = range(nw)
                rows = [g * L + j for j in js]
                tjs = [splat_at(t16, j) for j in js]
                e = [[] for _ in ws]
                cume = [[] for _ in ws]
                for i in range(NV):
                    for w in ws:
                        ei = jnp.exp(vin[p, rows[w], pl.ds(i * L, L)])
                        e[w].append(ei)
                        cume[w].append(plsc.cumsum(ei))
                prefv = [[] for _ in ws]
                pref = [jnp.zeros((L,), jnp.float32)] * nw
                for i in range(NV):
                    for w in ws:
                        prefv[w].append(pref[w])
                        pref[w] = pref[w] + splat_last(cume[w][i])
                invs = [1.0 / pref[w] for w in ws]
                cnt = [jnp.zeros((L,), jnp.int32)] * nw
                for i in range(NV):
                    for w in ws:
                        dtbuf[p, rows[w], pl.ds(i * L, L)] = e[w][i] * invs[w]
                        taui = (cume[w][i] + prefv[w][i]) * invs[w]
                        taubuf[js[w], pl.ds(i * L, L)] = taui
                        m = taui < tjs[w]
                        if i == NV - 1:
                            m = m & (lane < L - 1)
                        cnt[w] = cnt[w] + m.astype(jnp.int32)
                for w in ws:
                    indj = splat_last(plsc.cumsum(cnt[w]))
                    ind_acc = ind_acc + jnp.where(lane == js[w], indj, 0)
                return ind_acc

            for b in (0, 3, 6, 9, 12):
                ind_acc = do_rows((b, b + 1, b + 2), ind_acc)
            ind_acc = do_rows((15,), ind_acc)
            rows16 = g * L + lane
            p16 = jnp.broadcast_to(p, (L,))
            dtind16 = plsc.load_gather(dtbuf, [p16, rows16, ind_acc])
            taunext16 = plsc.load_gather(taubuf, [lane, ind_acc])
            indbuf[p, pl.ds(g * L, L)] = ind_acc
            dtindbuf[p, pl.ds(g * L, L)] = dtind16
            taunextbuf[p, pl.ds(g * L, L)] = taunext16
            tauindbuf[p, pl.ds(g * L, L)] = taunext16 - dtind16
            return gcarry

        lax.fori_loop(0, CHUNK // L, group_body, 0)

        for cp in out_copies(ci, p):
            cp.start()

        @pl.when(ci + 2 < NCHUNKS)
        def _():
            for cp in in_copies(ci + 2, p):
                cp.start()

        return carry

    lax.fori_loop(0, NCHUNKS, chunk_body, 0)

    for cp in out_copies(NCHUNKS - 2, 0):
        cp.wait()
    for cp in out_copies(NCHUNKS - 1, 1):
        cp.wait()


@jax.jit
def kernel(t, z):
    n = t.shape[0]
    mesh = plsc.VectorSubcoreMesh(core_axis_name="c", subcore_axis_name="s")
    out_type = (
        jax.ShapeDtypeStruct((n,), jnp.int32),
        jax.ShapeDtypeStruct((n, NUM_INTERVALS), jnp.float32),
        jax.ShapeDtypeStruct((n,), jnp.float32),
        jax.ShapeDtypeStruct((n,), jnp.float32),
        jax.ShapeDtypeStruct((n,), jnp.float32),
    )
    scratch = [
        pltpu.VMEM((2, CHUNK, NUM_INTERVALS), jnp.float32),
        pltpu.VMEM((2, CHUNK), jnp.float32),
        pltpu.VMEM((2, CHUNK, NUM_INTERVALS), jnp.float32),
        pltpu.VMEM((L, NUM_INTERVALS), jnp.float32),
        pltpu.VMEM((2, CHUNK), jnp.int32),
        pltpu.VMEM((2, CHUNK), jnp.float32),
        pltpu.VMEM((2, CHUNK), jnp.float32),
        pltpu.VMEM((2, CHUNK), jnp.float32),
        pltpu.SemaphoreType.DMA((2,)),
        pltpu.SemaphoreType.DMA((2,)),
    ]
    ind, dt, dt_ind, tau_ind, tau_next = pl.kernel(
        _sc_body,
        out_type=out_type,
        mesh=mesh,
        scratch_types=scratch,
        compiler_params=pltpu.CompilerParams(needs_layout_passes=False),
    )(t, z)
    z0 = z[:, :D_FEAT]
    return (ind, dt, dt_ind, tau_ind, tau_next, z0)
